# baseline (speedup 1.0000x reference)
import functools

import jax
import jax.numpy as jnp
from jax import lax
from jax.experimental import pallas as pl
from jax.experimental.pallas import tpu as pltpu
from jax.experimental.pallas import tpu_sc as plsc

B, S, D = 2, 2048, 1024
E, K, DFF = 8, 2, 2048
T = B * S
CAP = T * K * 5 // (E * 4)
NROW = E * CAP
DUMMY = NROW
NBUF = NROW + 8

NC, NS, L = 2, 16, 16
NW = NC * NS
TW = T // NW
CH = 32
NCH = TW // CH

TT = 512
FJ_A = 2
FB_A = DFF // FJ_A
FJ_E = 4
FB_E = DFF // FJ_E
NEG = -1e30


def _dense_body(x_ref, w1_ref, b1_ref, w2_ref, b2_ref, wgr_ref, mb_ref,
                xf_ref, meta_ref):
    j = pl.program_id(1)
    xb = x_ref[...]
    hp = jax.lax.dot_general(xb.astype(jnp.bfloat16), w1_ref[...],
                             (((1,), (0,)), ((), ())),
                             preferred_element_type=jnp.float32)
    h = jnp.maximum(hp + b1_ref[...], 0.0)
    part = jax.lax.dot_general(h.astype(jnp.bfloat16), w2_ref[...],
                               (((1,), (0,)), ((), ())),
                               preferred_element_type=jnp.float32)

    @pl.when(j == 0)
    def _():
        xf_ref[...] = part + b2_ref[...]
        m = xb @ wgr_ref[...] + mb_ref[...]
        lanes = lax.broadcasted_iota(jnp.int32, m.shape, 1)
        gate = 1.0 / (1.0 + jnp.exp(-m))
        meta_ref[...] = jnp.where(lanes == 0, gate, m)

    @pl.when(j != 0)
    def _():
        xf_ref[...] += part


def _dense_pass(x2d, w1, b1, w2, b2, wgr, mb):
    return pl.pallas_call(
        _dense_body,
        grid=(T // TT, FJ_A),
        in_specs=[
            pl.BlockSpec((TT, D), lambda t, j: (t, 0)),
            pl.BlockSpec((D, FB_A), lambda t, j: (0, j)),
            pl.BlockSpec((1, FB_A), lambda t, j: (0, j)),
            pl.BlockSpec((FB_A, D), lambda t, j: (j, 0)),
            pl.BlockSpec((1, D), lambda t, j: (0, 0)),
            pl.BlockSpec((D, 16), lambda t, j: (0, 0)),
            pl.BlockSpec((1, 16), lambda t, j: (0, 0)),
        ],
        out_specs=[
            pl.BlockSpec((TT, D), lambda t, j: (t, 0)),
            pl.BlockSpec((TT, 16), lambda t, j: (t, 0)),
        ],
        out_shape=[
            jax.ShapeDtypeStruct((T, D), jnp.float32),
            jax.ShapeDtypeStruct((T, 16), jnp.float32),
        ],
    )(x2d, w1, b1, w2, b2, wgr, mb)


def _route_body(meta_ref, iout_ref, fout_ref, scal_ref):
    CHB = 128
    NCHB = T // CHB
    r2 = lax.broadcasted_iota(jnp.int32, (CHB, CHB), 0)
    c2 = lax.broadcasted_iota(jnp.int32, (CHB, CHB), 1)
    tril_s = (c2 < r2).astype(jnp.float32)

    def step(c, carry):
        run, psum, gsum = carry
        m = meta_ref[pl.ds(c * CHB, CHB), :]
        lanes = lax.broadcasted_iota(jnp.int32, m.shape, 1)
        valid = (lanes >= 1) & (lanes <= E)
        ml = jnp.where(valid, m, NEG)
        mx = jnp.max(ml, axis=-1, keepdims=True)
        p = jnp.exp(ml - mx) * valid.astype(jnp.float32)
        p = p / jnp.sum(p, axis=-1, keepdims=True)

        v0 = jnp.max(p, axis=-1, keepdims=True)
        i0 = jnp.min(jnp.where((p == v0) & valid, lanes, 99), axis=-1, keepdims=True)
        p2 = jnp.where(lanes == i0, -1.0, p)
        v1 = jnp.max(p2, axis=-1, keepdims=True)
        i1 = jnp.min(jnp.where((p2 == v1) & valid, lanes, 99), axis=-1, keepdims=True)

        oh = ((lanes == i0) | (lanes == i1)).astype(jnp.float32)
        cex = jax.lax.dot_general(tril_s, oh, (((1,), (0,)), ((), ())),
                                  preferred_element_type=jnp.float32) + run
        pos0 = jnp.sum(cex * (lanes == i0), axis=-1, keepdims=True)
        pos1 = jnp.sum(cex * (lanes == i1), axis=-1, keepdims=True)

        run = run + jnp.sum(oh, axis=0, keepdims=True)
        psum = psum + jnp.sum(p, axis=0, keepdims=True)
        gsum = gsum + jnp.sum(m[:, 0:1])

        e0 = i0 - 1
        e1 = i1 - 1
        p0i = pos0.astype(jnp.int32)
        p1i = pos1.astype(jnp.int32)
        k0 = p0i < CAP
        k1 = p1i < CAP
        s0 = e0 * CAP + p0i
        s1 = e1 * CAP + p1i
        d0 = jnp.where(k0, s0, DUMMY)
        d1 = jnp.where(k1, s1, DUMMY)
        c0 = jnp.where(k0, s0, 0)
        c1 = jnp.where(k1, s1, 0)
        w0 = v0 * k0.astype(jnp.float32)
        w1 = v1 * k1.astype(jnp.float32)

        l8 = lax.broadcasted_iota(jnp.int32, (CHB, 8), 1)
        ci = jnp.where(l8 == 0, d0,
             jnp.where(l8 == 1, d1,
             jnp.where(l8 == 2, c0,
             jnp.where(l8 == 3, c1, 0))))
        cf = jnp.where(l8 == 0, w0, jnp.where(l8 == 1, w1, 0.0))
        iout_ref[pl.ds(c * CHB, CHB), :] = ci
        fout_ref[pl.ds(c * CHB, CHB), :] = cf
        return run, psum, gsum

    init = (jnp.zeros((1, 16), jnp.float32), jnp.zeros((1, 16), jnp.float32),
            jnp.zeros((), jnp.float32))
    run, psum, gsum = lax.fori_loop(0, NCHB, step, init)

    me = psum / jnp.float32(T)
    l_aux = jnp.float32(E) * jnp.sum(me * run / jnp.float32(T))
    sr = lax.broadcasted_iota(jnp.int32, (8, 16), 0)
    sl = lax.broadcasted_iota(jnp.int32, (8, 16), 1)
    counts_b = jnp.broadcast_to(run, (8, 16))
    scal_ref[...] = jnp.where((sr == 0) & (sl == 0), l_aux,
                    jnp.where((sr == 0) & (sl == 1), gsum,
                    jnp.where(sr == 1, counts_b, 0.0)))


def _route_pass(meta):
    return pl.pallas_call(
        _route_body,
        out_shape=[
            jax.ShapeDtypeStruct((T, 8), jnp.int32),
            jax.ShapeDtypeStruct((T, 8), jnp.float32),
            jax.ShapeDtypeStruct((8, 16), jnp.float32),
        ],
    )(meta)


@functools.cache
def _sc_dispatch():
    mesh = plsc.VectorSubcoreMesh(core_axis_name="c", subcore_axis_name="s",
                                  num_cores=NC, num_subcores=NS)

    @functools.partial(
        pl.kernel,
        out_type=jax.ShapeDtypeStruct((NBUF, D), jnp.float32),
        mesh=mesh,
        scratch_types=[
            pltpu.VMEM((CH, D), jnp.float32),
            pltpu.VMEM((NCH, CH), jnp.int32),
            pltpu.VMEM((NCH, CH), jnp.int32),
            pltpu.SemaphoreType.DMA,
            pltpu.SemaphoreType.DMA,
        ],
        compiler_params=pltpu.CompilerParams(needs_layout_passes=False),
    )
    def body(x_hbm, d0_hbm, d1_hbm, buf_hbm, rows_v, i0_v, i1_v, sem0, sem1):
        wid = lax.axis_index("s") * NC + lax.axis_index("c")
        base = wid * TW
        pltpu.sync_copy(d0_hbm.at[wid], i0_v)
        pltpu.sync_copy(d1_hbm.at[wid], i1_v)
        for j in range(NCH):
            pltpu.sync_copy(x_hbm.at[pl.ds(base + j * CH, CH)], rows_v)
            c0 = pltpu.async_copy(rows_v, buf_hbm.at[i0_v.at[j]], sem0)
            c1 = pltpu.async_copy(rows_v, buf_hbm.at[i1_v.at[j]], sem1)
            c0.wait()
            c1.wait()

    return body


def _dispatch(x2d, d0, d1):
    return _sc_dispatch()(x2d, d0, d1)


def _expert_body(counts_ref, buf_ref, w1_ref, b1_ref, w2_ref, b2_ref, out_ref):
    e = pl.program_id(0)
    j = pl.program_id(1)
    cnt = counts_ref[e]
    riota = lax.broadcasted_iota(jnp.int32, (CAP, 1), 0)
    xm = jnp.where(riota < cnt, buf_ref[...], 0.0)
    hp = jax.lax.dot_general(xm.astype(jnp.bfloat16), w1_ref[0],
                             (((1,), (0,)), ((), ())),
                             preferred_element_type=jnp.float32)
    h = jnp.maximum(hp + b1_ref[0], 0.0)
    part = jax.lax.dot_general(h.astype(jnp.bfloat16), w2_ref[0],
                               (((1,), (0,)), ((), ())),
                               preferred_element_type=jnp.float32)

    @pl.when(j == 0)
    def _():
        out_ref[...] = part + b2_ref[0]

    @pl.when(j != 0)
    def _():
        out_ref[...] += part


def _expert_pass(counts, bufp, ew1, eb1, ew2, eb2):
    return pl.pallas_call(
        _expert_body,
        grid=(E, FJ_E),
        in_specs=[
            pl.BlockSpec(memory_space=pltpu.SMEM),
            pl.BlockSpec((CAP, D), lambda e, j: (e, 0)),
            pl.BlockSpec((1, D, FB_E), lambda e, j: (e, 0, j)),
            pl.BlockSpec((1, 1, FB_E), lambda e, j: (e, 0, j)),
            pl.BlockSpec((1, FB_E, D), lambda e, j: (e, j, 0)),
            pl.BlockSpec((1, 1, D), lambda e, j: (e, 0, 0)),
        ],
        out_specs=pl.BlockSpec((CAP, D), lambda e, j: (e, 0)),
        out_shape=jax.ShapeDtypeStruct((NROW, D), jnp.float32),
    )(counts, bufp, ew1, eb1[:, None, :], ew2, eb2[:, None, :])


@functools.cache
def _sc_combine():
    mesh = plsc.VectorSubcoreMesh(core_axis_name="c", subcore_axis_name="s",
                                  num_cores=NC, num_subcores=NS)

    @functools.partial(
        pl.kernel,
        out_type=jax.ShapeDtypeStruct((T, D), jnp.float32),
        mesh=mesh,
        scratch_types=[
            pltpu.VMEM((CH, D), jnp.float32),
            pltpu.VMEM((CH, D), jnp.float32),
            pltpu.VMEM((CH, D), jnp.float32),
            pltpu.VMEM((NCH, CH), jnp.int32),
            pltpu.VMEM((NCH, CH), jnp.int32),
            pltpu.VMEM((TW,), jnp.float32),
            pltpu.VMEM((TW,), jnp.float32),
            pltpu.VMEM((TW,), jnp.float32),
            pltpu.SemaphoreType.DMA,
            pltpu.SemaphoreType.DMA,
        ],
        compiler_params=pltpu.CompilerParams(needs_layout_passes=False),
    )
    def body(outb_hbm, c0_hbm, c1_hbm, w0_hbm, w1_hbm, g_hbm, xf_hbm, y_hbm,
             r0_v, r1_v, xf_v, i0_v, i1_v, w0_v, w1_v, g_v, sem0, sem1):
        wid = lax.axis_index("s") * NC + lax.axis_index("c")
        base = wid * TW
        pltpu.sync_copy(c0_hbm.at[wid], i0_v)
        pltpu.sync_copy(c1_hbm.at[wid], i1_v)
        pltpu.sync_copy(w0_hbm.at[wid], w0_v)
        pltpu.sync_copy(w1_hbm.at[wid], w1_v)
        pltpu.sync_copy(g_hbm.at[wid], g_v)
        for j in range(NCH):
            a0 = pltpu.async_copy(outb_hbm.at[i0_v.at[j]], r0_v, sem0)
            a1 = pltpu.async_copy(outb_hbm.at[i1_v.at[j]], r1_v, sem1)
            pltpu.sync_copy(xf_hbm.at[pl.ds(base + j * CH, CH)], xf_v)
            a0.wait()
            a1.wait()

            def token(tl, _):
                gt = jnp.full((L,), j * CH + tl, jnp.int32)
                w0s = plsc.load_gather(w0_v, [gt])
                w1s = plsc.load_gather(w1_v, [gt])
                gs = plsc.load_gather(g_v, [gt])
                for gi in range(D // L):
                    sl = pl.ds(gi * L, L)
                    r0 = r0_v[tl, sl]
                    r1 = r1_v[tl, sl]
                    xf = xf_v[tl, sl]
                    moe = r0 * w0s + r1 * w1s
                    xf_v[tl, sl] = xf + gs * (moe - xf)
                return 0

            lax.fori_loop(0, CH, token, 0)
            pltpu.sync_copy(xf_v, y_hbm.at[pl.ds(base + j * CH, CH)])

    return body


def _combine(outb, c0, c1, w0, w1, g, xf):
    return _sc_combine()(outb, c0, c1, w0, w1, g, xf)


def kernel(x, wg_w, wg_b, ffn_w1, ffn_b1, ffn_w2, ffn_b2, router_w, ew1, eb1, ew2, eb2):
    x2d = x.reshape(T, D)
    wgr = jnp.pad(jnp.concatenate([wg_w, router_w], axis=1), ((0, 0), (0, 16 - 1 - E)))
    mb = jnp.pad(wg_b[None, :], ((0, 0), (0, 15)))
    b1 = ffn_b1[None, :]
    b2 = ffn_b2[None, :]

    xf, meta = _dense_pass(x2d, ffn_w1.astype(jnp.bfloat16), b1,
                           ffn_w2.astype(jnp.bfloat16), b2, wgr, mb)
    iout, fout, scal = _route_pass(meta)

    d0 = iout[:, 0].reshape(NW, NCH, CH)
    d1 = iout[:, 1].reshape(NW, NCH, CH)
    c0 = iout[:, 2].reshape(NW, NCH, CH)
    c1 = iout[:, 3].reshape(NW, NCH, CH)
    w0 = fout[:, 0].reshape(NW, TW)
    w1 = fout[:, 1].reshape(NW, TW)
    g = meta[:, 0].reshape(NW, TW)
    counts = scal[1, 1:1 + E].astype(jnp.int32)

    bufp = _dispatch(x2d, d0, d1)
    outb = _expert_pass(counts, bufp, ew1.astype(jnp.bfloat16), eb1,
                        ew2.astype(jnp.bfloat16), eb2)
    y = _combine(outb, c0, c1, w0, w1, g, xf)

    x_out = y.reshape(B, S, D)
    l_aux = scal[0, 0]
    used = scal[0, 1]
    return (x_out, l_aux, used, jnp.float32(T))


# speedup vs baseline: 1.1558x; 1.1558x over previous
import functools

import jax
import jax.numpy as jnp
from jax import lax
from jax.experimental import pallas as pl
from jax.experimental.pallas import tpu as pltpu
from jax.experimental.pallas import tpu_sc as plsc

B, S, D = 2, 2048, 1024
E, K, DFF = 8, 2, 2048
T = B * S
CAP = T * K * 5 // (E * 4)
NROW = E * CAP
DUMMY = NROW
NBUF = NROW + 8

NC, NS, L = 2, 16, 16
NW = NC * NS
TW = T // NW
CH = 32
NCH = TW // CH

TT = 512
FJ_A = 2
FB_A = DFF // FJ_A
FJ_E = 4
FB_E = DFF // FJ_E
NEG = -1e30


def _meta_body(x_ref, wgr_ref, mb_ref, meta_ref):
    m = x_ref[...] @ wgr_ref[...] + mb_ref[...]
    lanes = lax.broadcasted_iota(jnp.int32, m.shape, 1)
    gate = 1.0 / (1.0 + jnp.exp(-m))
    meta_ref[...] = jnp.where(lanes == 0, gate, m)


def _meta_pass(x2d, wgr, mb):
    return pl.pallas_call(
        _meta_body,
        grid=(4,),
        in_specs=[
            pl.BlockSpec((T // 4, D), lambda t: (t, 0)),
            pl.BlockSpec((D, 16), lambda t: (0, 0)),
            pl.BlockSpec((1, 16), lambda t: (0, 0)),
        ],
        out_specs=pl.BlockSpec((T // 4, 16), lambda t: (t, 0)),
        out_shape=jax.ShapeDtypeStruct((T, 16), jnp.float32),
    )(x2d, wgr, mb)


def _dense_body(x_ref, w1_ref, b1_ref, w2_ref, b2_ref, xf_ref):
    j = pl.program_id(1)
    xb = x_ref[...]
    h = jnp.maximum(xb @ w1_ref[...] + b1_ref[...], 0.0)
    part = jax.lax.dot_general(h, w2_ref[...], (((1,), (0,)), ((), ())),
                               preferred_element_type=jnp.float32)

    @pl.when(j == 0)
    def _():
        xf_ref[...] = part + b2_ref[...]

    @pl.when(j != 0)
    def _():
        xf_ref[...] += part


def _dense_pass(x2d, w1, b1, w2, b2):
    return pl.pallas_call(
        _dense_body,
        grid=(T // TT, FJ_A),
        in_specs=[
            pl.BlockSpec((TT, D), lambda t, j: (t, 0)),
            pl.BlockSpec((D, FB_A), lambda t, j: (0, j)),
            pl.BlockSpec((1, FB_A), lambda t, j: (0, j)),
            pl.BlockSpec((FB_A, D), lambda t, j: (j, 0)),
            pl.BlockSpec((1, D), lambda t, j: (0, 0)),
        ],
        out_specs=pl.BlockSpec((TT, D), lambda t, j: (t, 0)),
        out_shape=jax.ShapeDtypeStruct((T, D), jnp.float32),
    )(x2d, w1, b1, w2, b2)


def _route_body(meta_ref, iout_ref, fout_ref, scal_ref):
    CHB = 128
    NCHB = T // CHB
    r2 = lax.broadcasted_iota(jnp.int32, (CHB, CHB), 0)
    c2 = lax.broadcasted_iota(jnp.int32, (CHB, CHB), 1)
    tril_s = (c2 < r2).astype(jnp.float32)

    def step(c, carry):
        run, psum, gsum = carry
        m = meta_ref[pl.ds(c * CHB, CHB), :]
        lanes = lax.broadcasted_iota(jnp.int32, m.shape, 1)
        valid = (lanes >= 1) & (lanes <= E)
        ml = jnp.where(valid, m, NEG)
        mx = jnp.max(ml, axis=-1, keepdims=True)
        p = jnp.exp(ml - mx) * valid.astype(jnp.float32)
        p = p / jnp.sum(p, axis=-1, keepdims=True)

        v0 = jnp.max(p, axis=-1, keepdims=True)
        i0 = jnp.min(jnp.where((p == v0) & valid, lanes, 99), axis=-1, keepdims=True)
        p2 = jnp.where(lanes == i0, -1.0, p)
        v1 = jnp.max(p2, axis=-1, keepdims=True)
        i1 = jnp.min(jnp.where((p2 == v1) & valid, lanes, 99), axis=-1, keepdims=True)

        oh = ((lanes == i0) | (lanes == i1)).astype(jnp.float32)
        cex = jax.lax.dot_general(tril_s, oh, (((1,), (0,)), ((), ())),
                                  preferred_element_type=jnp.float32) + run
        pos0 = jnp.sum(cex * (lanes == i0), axis=-1, keepdims=True)
        pos1 = jnp.sum(cex * (lanes == i1), axis=-1, keepdims=True)

        run = run + jnp.sum(oh, axis=0, keepdims=True)
        psum = psum + jnp.sum(p, axis=0, keepdims=True)
        gsum = gsum + jnp.sum(m[:, 0:1])

        e0 = i0 - 1
        e1 = i1 - 1
        p0i = pos0.astype(jnp.int32)
        p1i = pos1.astype(jnp.int32)
        k0 = p0i < CAP
        k1 = p1i < CAP
        s0 = e0 * CAP + p0i
        s1 = e1 * CAP + p1i
        d0 = jnp.where(k0, s0, DUMMY)
        d1 = jnp.where(k1, s1, DUMMY)
        c0 = jnp.where(k0, s0, 0)
        c1 = jnp.where(k1, s1, 0)
        w0 = v0 * k0.astype(jnp.float32)
        w1 = v1 * k1.astype(jnp.float32)

        l8 = lax.broadcasted_iota(jnp.int32, (CHB, 8), 1)
        ci = jnp.where(l8 == 0, d0,
             jnp.where(l8 == 1, d1,
             jnp.where(l8 == 2, c0,
             jnp.where(l8 == 3, c1, 0))))
        cf = jnp.where(l8 == 0, w0, jnp.where(l8 == 1, w1, 0.0))
        iout_ref[pl.ds(c * CHB, CHB), :] = ci
        fout_ref[pl.ds(c * CHB, CHB), :] = cf
        return run, psum, gsum

    init = (jnp.zeros((1, 16), jnp.float32), jnp.zeros((1, 16), jnp.float32),
            jnp.zeros((), jnp.float32))
    run, psum, gsum = lax.fori_loop(0, NCHB, step, init)

    me = psum / jnp.float32(T)
    l_aux = jnp.float32(E) * jnp.sum(me * run / jnp.float32(T))
    sr = lax.broadcasted_iota(jnp.int32, (8, 16), 0)
    sl = lax.broadcasted_iota(jnp.int32, (8, 16), 1)
    counts_b = jnp.broadcast_to(run, (8, 16))
    scal_ref[...] = jnp.where((sr == 0) & (sl == 0), l_aux,
                    jnp.where((sr == 0) & (sl == 1), gsum,
                    jnp.where(sr == 1, counts_b, 0.0)))


def _route_pass(meta):
    return pl.pallas_call(
        _route_body,
        out_shape=[
            jax.ShapeDtypeStruct((T, 8), jnp.int32),
            jax.ShapeDtypeStruct((T, 8), jnp.float32),
            jax.ShapeDtypeStruct((8, 16), jnp.float32),
        ],
    )(meta)


@functools.cache
def _sc_dispatch():
    mesh = plsc.VectorSubcoreMesh(core_axis_name="c", subcore_axis_name="s",
                                  num_cores=NC, num_subcores=NS)

    @functools.partial(
        pl.kernel,
        out_type=jax.ShapeDtypeStruct((NBUF, D), jnp.float32),
        mesh=mesh,
        scratch_types=[
            pltpu.VMEM((CH, D), jnp.float32),
            pltpu.VMEM((NCH, CH), jnp.int32),
            pltpu.VMEM((NCH, CH), jnp.int32),
            pltpu.SemaphoreType.DMA,
            pltpu.SemaphoreType.DMA,
        ],
        compiler_params=pltpu.CompilerParams(needs_layout_passes=False),
    )
    def body(x_hbm, d0_hbm, d1_hbm, buf_hbm, rows_v, i0_v, i1_v, sem0, sem1):
        wid = lax.axis_index("s") * NC + lax.axis_index("c")
        base = wid * TW
        pltpu.sync_copy(d0_hbm.at[wid], i0_v)
        pltpu.sync_copy(d1_hbm.at[wid], i1_v)
        for j in range(NCH):
            pltpu.sync_copy(x_hbm.at[pl.ds(base + j * CH, CH)], rows_v)
            c0 = pltpu.async_copy(rows_v, buf_hbm.at[i0_v.at[j]], sem0)
            c1 = pltpu.async_copy(rows_v, buf_hbm.at[i1_v.at[j]], sem1)
            c0.wait()
            c1.wait()

    return body


def _dispatch(x2d, d0, d1):
    return _sc_dispatch()(x2d, d0, d1)


def _expert_body(counts_ref, buf_ref, w1_ref, b1_ref, w2_ref, b2_ref, out_ref):
    e = pl.program_id(0)
    j = pl.program_id(1)
    cnt = counts_ref[e]
    riota = lax.broadcasted_iota(jnp.int32, (CAP, 1), 0)
    xm = jnp.where(riota < cnt, buf_ref[...], 0.0)
    h = jnp.maximum(xm @ w1_ref[0] + b1_ref[0], 0.0)
    part = jax.lax.dot_general(h, w2_ref[0], (((1,), (0,)), ((), ())),
                               preferred_element_type=jnp.float32)

    @pl.when(j == 0)
    def _():
        out_ref[...] = part + b2_ref[0]

    @pl.when(j != 0)
    def _():
        out_ref[...] += part


def _expert_pass(counts, bufp, ew1, eb1, ew2, eb2):
    return pl.pallas_call(
        _expert_body,
        grid=(E, FJ_E),
        in_specs=[
            pl.BlockSpec(memory_space=pltpu.SMEM),
            pl.BlockSpec((CAP, D), lambda e, j: (e, 0)),
            pl.BlockSpec((1, D, FB_E), lambda e, j: (e, 0, j)),
            pl.BlockSpec((1, 1, FB_E), lambda e, j: (e, 0, j)),
            pl.BlockSpec((1, FB_E, D), lambda e, j: (e, j, 0)),
            pl.BlockSpec((1, 1, D), lambda e, j: (e, 0, 0)),
        ],
        out_specs=pl.BlockSpec((CAP, D), lambda e, j: (e, 0)),
        out_shape=jax.ShapeDtypeStruct((NROW, D), jnp.float32),
    )(counts, bufp, ew1, eb1[:, None, :], ew2, eb2[:, None, :])


@functools.cache
def _sc_combine():
    mesh = plsc.VectorSubcoreMesh(core_axis_name="c", subcore_axis_name="s",
                                  num_cores=NC, num_subcores=NS)

    @functools.partial(
        pl.kernel,
        out_type=jax.ShapeDtypeStruct((T, D), jnp.float32),
        mesh=mesh,
        scratch_types=[
            pltpu.VMEM((CH, D), jnp.float32),
            pltpu.VMEM((CH, D), jnp.float32),
            pltpu.VMEM((CH, D), jnp.float32),
            pltpu.VMEM((NCH, CH), jnp.int32),
            pltpu.VMEM((NCH, CH), jnp.int32),
            pltpu.VMEM((TW,), jnp.float32),
            pltpu.VMEM((TW,), jnp.float32),
            pltpu.VMEM((TW,), jnp.float32),
            pltpu.SemaphoreType.DMA,
            pltpu.SemaphoreType.DMA,
        ],
        compiler_params=pltpu.CompilerParams(needs_layout_passes=False),
    )
    def body(outb_hbm, c0_hbm, c1_hbm, w0_hbm, w1_hbm, g_hbm, xf_hbm, y_hbm,
             r0_v, r1_v, xf_v, i0_v, i1_v, w0_v, w1_v, g_v, sem0, sem1):
        wid = lax.axis_index("s") * NC + lax.axis_index("c")
        base = wid * TW
        pltpu.sync_copy(c0_hbm.at[wid], i0_v)
        pltpu.sync_copy(c1_hbm.at[wid], i1_v)
        pltpu.sync_copy(w0_hbm.at[wid], w0_v)
        pltpu.sync_copy(w1_hbm.at[wid], w1_v)
        pltpu.sync_copy(g_hbm.at[wid], g_v)
        for j in range(NCH):
            a0 = pltpu.async_copy(outb_hbm.at[i0_v.at[j]], r0_v, sem0)
            a1 = pltpu.async_copy(outb_hbm.at[i1_v.at[j]], r1_v, sem1)
            pltpu.sync_copy(xf_hbm.at[pl.ds(base + j * CH, CH)], xf_v)
            a0.wait()
            a1.wait()

            def token(tl, _):
                gt = jnp.full((L,), j * CH + tl, jnp.int32)
                w0s = plsc.load_gather(w0_v, [gt])
                w1s = plsc.load_gather(w1_v, [gt])
                gs = plsc.load_gather(g_v, [gt])
                for gi in range(D // L):
                    sl = pl.ds(gi * L, L)
                    r0 = r0_v[tl, sl]
                    r1 = r1_v[tl, sl]
                    xf = xf_v[tl, sl]
                    moe = r0 * w0s + r1 * w1s
                    xf_v[tl, sl] = xf + gs * (moe - xf)
                return 0

            lax.fori_loop(0, CH, token, 0)
            pltpu.sync_copy(xf_v, y_hbm.at[pl.ds(base + j * CH, CH)])

    return body


def _combine(outb, c0, c1, w0, w1, g, xf):
    return _sc_combine()(outb, c0, c1, w0, w1, g, xf)


def kernel(x, wg_w, wg_b, ffn_w1, ffn_b1, ffn_w2, ffn_b2, router_w, ew1, eb1, ew2, eb2):
    x2d = x.reshape(T, D)
    wgr = jnp.pad(jnp.concatenate([wg_w, router_w], axis=1), ((0, 0), (0, 16 - 1 - E)))
    mb = jnp.pad(wg_b[None, :], ((0, 0), (0, 15)))
    b1 = ffn_b1[None, :]
    b2 = ffn_b2[None, :]

    meta = _meta_pass(x2d, wgr, mb)
    iout, fout, scal = _route_pass(meta)
    xf = _dense_pass(x2d, ffn_w1, b1, ffn_w2, b2)

    d0 = iout[:, 0].reshape(NW, NCH, CH)
    d1 = iout[:, 1].reshape(NW, NCH, CH)
    c0 = iout[:, 2].reshape(NW, NCH, CH)
    c1 = iout[:, 3].reshape(NW, NCH, CH)
    w0 = fout[:, 0].reshape(NW, TW)
    w1 = fout[:, 1].reshape(NW, TW)
    g = meta[:, 0].reshape(NW, TW)
    counts = scal[1, 1:1 + E].astype(jnp.int32)

    bufp = _dispatch(x2d, d0, d1)
    outb = _expert_pass(counts, bufp, ew1, eb1, ew2, eb2)
    y = _combine(outb, c0, c1, w0, w1, g, xf)

    x_out = y.reshape(B, S, D)
    l_aux = scal[0, 0]
    used = scal[0, 1]
    return (x_out, l_aux, used, jnp.float32(T))


# speedup vs baseline: 1.1639x; 1.0070x over previous
import functools

import jax
import jax.numpy as jnp
from jax import lax
from jax.experimental import pallas as pl
from jax.experimental.pallas import tpu as pltpu
from jax.experimental.pallas import tpu_sc as plsc

B, S, D = 2, 2048, 1024
E, K, DFF = 8, 2, 2048
T = B * S
CAP = T * K * 5 // (E * 4)
NROW = E * CAP
DUMMY = NROW
NBUF = NROW + 8

NC, NS, L = 2, 16, 16
NW = NC * NS
TW = T // NW
CH = 32
NCH = TW // CH

TT = 512
FJ_A = 2
FB_A = DFF // FJ_A
FJ_E = 4
FB_E = DFF // FJ_E
NEG = -1e30


def _dense_body(x_ref, w1_ref, b1_ref, w2_ref, b2_ref, wgr_ref, mb_ref,
                xf_ref, meta_ref):
    j = pl.program_id(1)
    xb = x_ref[...]
    h = jnp.maximum(xb @ w1_ref[...] + b1_ref[...], 0.0)
    part = jax.lax.dot_general(h, w2_ref[...], (((1,), (0,)), ((), ())),
                               preferred_element_type=jnp.float32)

    @pl.when(j == 0)
    def _():
        xf_ref[...] = part + b2_ref[...]
        m = xb @ wgr_ref[...] + mb_ref[...]
        lanes = lax.broadcasted_iota(jnp.int32, m.shape, 1)
        gate = 1.0 / (1.0 + jnp.exp(-m))
        meta_ref[...] = jnp.where(lanes == 0, gate, m)

    @pl.when(j != 0)
    def _():
        xf_ref[...] += part


def _dense_pass(x2d, w1, b1, w2, b2, wgr, mb):
    return pl.pallas_call(
        _dense_body,
        grid=(T // TT, FJ_A),
        in_specs=[
            pl.BlockSpec((TT, D), lambda t, j: (t, 0)),
            pl.BlockSpec((D, FB_A), lambda t, j: (0, j)),
            pl.BlockSpec((1, FB_A), lambda t, j: (0, j)),
            pl.BlockSpec((FB_A, D), lambda t, j: (j, 0)),
            pl.BlockSpec((1, D), lambda t, j: (0, 0)),
            pl.BlockSpec((D, 16), lambda t, j: (0, 0)),
            pl.BlockSpec((1, 16), lambda t, j: (0, 0)),
        ],
        out_specs=[
            pl.BlockSpec((TT, D), lambda t, j: (t, 0)),
            pl.BlockSpec((TT, 16), lambda t, j: (t, 0)),
        ],
        out_shape=[
            jax.ShapeDtypeStruct((T, D), jnp.float32),
            jax.ShapeDtypeStruct((T, 16), jnp.float32),
        ],
    )(x2d, w1, b1, w2, b2, wgr, mb)


def _route_body(meta_ref, iout_ref, fout_ref, scal_ref):
    CHB = 128
    NCHB = T // CHB
    r2 = lax.broadcasted_iota(jnp.int32, (CHB, CHB), 0)
    c2 = lax.broadcasted_iota(jnp.int32, (CHB, CHB), 1)
    tril_s = (c2 < r2).astype(jnp.float32)

    def step(c, carry):
        run, psum, gsum = carry
        m = meta_ref[pl.ds(c * CHB, CHB), :]
        lanes = lax.broadcasted_iota(jnp.int32, m.shape, 1)
        valid = (lanes >= 1) & (lanes <= E)
        ml = jnp.where(valid, m, NEG)
        mx = jnp.max(ml, axis=-1, keepdims=True)
        p = jnp.exp(ml - mx) * valid.astype(jnp.float32)
        p = p / jnp.sum(p, axis=-1, keepdims=True)

        v0 = jnp.max(p, axis=-1, keepdims=True)
        i0 = jnp.min(jnp.where((p == v0) & valid, lanes, 99), axis=-1, keepdims=True)
        p2 = jnp.where(lanes == i0, -1.0, p)
        v1 = jnp.max(p2, axis=-1, keepdims=True)
        i1 = jnp.min(jnp.where((p2 == v1) & valid, lanes, 99), axis=-1, keepdims=True)

        oh = ((lanes == i0) | (lanes == i1)).astype(jnp.float32)
        cex = jax.lax.dot_general(tril_s, oh, (((1,), (0,)), ((), ())),
                                  preferred_element_type=jnp.float32) + run
        pos0 = jnp.sum(cex * (lanes == i0), axis=-1, keepdims=True)
        pos1 = jnp.sum(cex * (lanes == i1), axis=-1, keepdims=True)

        run = run + jnp.sum(oh, axis=0, keepdims=True)
        psum = psum + jnp.sum(p, axis=0, keepdims=True)
        gsum = gsum + jnp.sum(m[:, 0:1])

        e0 = i0 - 1
        e1 = i1 - 1
        p0i = pos0.astype(jnp.int32)
        p1i = pos1.astype(jnp.int32)
        k0 = p0i < CAP
        k1 = p1i < CAP
        s0 = e0 * CAP + p0i
        s1 = e1 * CAP + p1i
        d0 = jnp.where(k0, s0, DUMMY)
        d1 = jnp.where(k1, s1, DUMMY)
        c0 = jnp.where(k0, s0, 0)
        c1 = jnp.where(k1, s1, 0)
        w0 = v0 * k0.astype(jnp.float32)
        w1 = v1 * k1.astype(jnp.float32)

        l8 = lax.broadcasted_iota(jnp.int32, (CHB, 8), 1)
        ci = jnp.where(l8 == 0, d0,
             jnp.where(l8 == 1, d1,
             jnp.where(l8 == 2, c0,
             jnp.where(l8 == 3, c1, 0))))
        cf = jnp.where(l8 == 0, w0, jnp.where(l8 == 1, w1, 0.0))
        iout_ref[pl.ds(c * CHB, CHB), :] = ci
        fout_ref[pl.ds(c * CHB, CHB), :] = cf
        return run, psum, gsum

    init = (jnp.zeros((1, 16), jnp.float32), jnp.zeros((1, 16), jnp.float32),
            jnp.zeros((), jnp.float32))
    run, psum, gsum = lax.fori_loop(0, NCHB, step, init)

    me = psum / jnp.float32(T)
    l_aux = jnp.float32(E) * jnp.sum(me * run / jnp.float32(T))
    sr = lax.broadcasted_iota(jnp.int32, (8, 16), 0)
    sl = lax.broadcasted_iota(jnp.int32, (8, 16), 1)
    counts_b = jnp.broadcast_to(run, (8, 16))
    scal_ref[...] = jnp.where((sr == 0) & (sl == 0), l_aux,
                    jnp.where((sr == 0) & (sl == 1), gsum,
                    jnp.where(sr == 1, counts_b, 0.0)))


def _route_pass(meta):
    return pl.pallas_call(
        _route_body,
        out_shape=[
            jax.ShapeDtypeStruct((T, 8), jnp.int32),
            jax.ShapeDtypeStruct((T, 8), jnp.float32),
            jax.ShapeDtypeStruct((8, 16), jnp.float32),
        ],
    )(meta)


@functools.cache
def _sc_dispatch():
    mesh = plsc.VectorSubcoreMesh(core_axis_name="c", subcore_axis_name="s",
                                  num_cores=NC, num_subcores=NS)

    @functools.partial(
        pl.kernel,
        out_type=jax.ShapeDtypeStruct((NBUF, D), jnp.float32),
        mesh=mesh,
        scratch_types=[
            pltpu.VMEM((CH, D), jnp.float32),
            pltpu.VMEM((NCH, CH), jnp.int32),
            pltpu.VMEM((NCH, CH), jnp.int32),
            pltpu.SemaphoreType.DMA,
            pltpu.SemaphoreType.DMA,
        ],
        compiler_params=pltpu.CompilerParams(needs_layout_passes=False),
    )
    def body(x_hbm, d0_hbm, d1_hbm, buf_hbm, rows_v, i0_v, i1_v, sem0, sem1):
        wid = lax.axis_index("s") * NC + lax.axis_index("c")
        base = wid * TW
        pltpu.sync_copy(d0_hbm.at[wid], i0_v)
        pltpu.sync_copy(d1_hbm.at[wid], i1_v)
        for j in range(NCH):
            pltpu.sync_copy(x_hbm.at[pl.ds(base + j * CH, CH)], rows_v)
            c0 = pltpu.async_copy(rows_v, buf_hbm.at[i0_v.at[j]], sem0)
            c1 = pltpu.async_copy(rows_v, buf_hbm.at[i1_v.at[j]], sem1)
            c0.wait()
            c1.wait()

    return body


def _dispatch(x2d, d0, d1):
    return _sc_dispatch()(x2d, d0, d1)


def _expert_body(counts_ref, buf_ref, w1_ref, b1_ref, w2_ref, b2_ref, out_ref):
    e = pl.program_id(0)
    j = pl.program_id(1)
    cnt = counts_ref[e]
    riota = lax.broadcasted_iota(jnp.int32, (CAP, 1), 0)
    xm = jnp.where(riota < cnt, buf_ref[...], 0.0)
    h = jnp.maximum(xm @ w1_ref[0] + b1_ref[0], 0.0)
    part = jax.lax.dot_general(h, w2_ref[0], (((1,), (0,)), ((), ())),
                               preferred_element_type=jnp.float32)

    @pl.when(j == 0)
    def _():
        out_ref[...] = part + b2_ref[0]

    @pl.when(j != 0)
    def _():
        out_ref[...] += part


def _expert_pass(counts, bufp, ew1, eb1, ew2, eb2):
    return pl.pallas_call(
        _expert_body,
        grid=(E, FJ_E),
        in_specs=[
            pl.BlockSpec(memory_space=pltpu.SMEM),
            pl.BlockSpec((CAP, D), lambda e, j: (e, 0)),
            pl.BlockSpec((1, D, FB_E), lambda e, j: (e, 0, j)),
            pl.BlockSpec((1, 1, FB_E), lambda e, j: (e, 0, j)),
            pl.BlockSpec((1, FB_E, D), lambda e, j: (e, j, 0)),
            pl.BlockSpec((1, 1, D), lambda e, j: (e, 0, 0)),
        ],
        out_specs=pl.BlockSpec((CAP, D), lambda e, j: (e, 0)),
        out_shape=jax.ShapeDtypeStruct((NROW, D), jnp.float32),
    )(counts, bufp, ew1, eb1[:, None, :], ew2, eb2[:, None, :])


@functools.cache
def _sc_combine():
    mesh = plsc.VectorSubcoreMesh(core_axis_name="c", subcore_axis_name="s",
                                  num_cores=NC, num_subcores=NS)

    @functools.partial(
        pl.kernel,
        out_type=jax.ShapeDtypeStruct((T, D), jnp.float32),
        mesh=mesh,
        scratch_types=[
            pltpu.VMEM((CH, D), jnp.float32),
            pltpu.VMEM((CH, D), jnp.float32),
            pltpu.VMEM((CH, D), jnp.float32),
            pltpu.VMEM((NCH, CH), jnp.int32),
            pltpu.VMEM((NCH, CH), jnp.int32),
            pltpu.VMEM((TW,), jnp.float32),
            pltpu.VMEM((TW,), jnp.float32),
            pltpu.VMEM((TW,), jnp.float32),
            pltpu.SemaphoreType.DMA,
            pltpu.SemaphoreType.DMA,
        ],
        compiler_params=pltpu.CompilerParams(needs_layout_passes=False),
    )
    def body(outb_hbm, c0_hbm, c1_hbm, w0_hbm, w1_hbm, g_hbm, xf_hbm, y_hbm,
             r0_v, r1_v, xf_v, i0_v, i1_v, w0_v, w1_v, g_v, sem0, sem1):
        wid = lax.axis_index("s") * NC + lax.axis_index("c")
        base = wid * TW
        pltpu.sync_copy(c0_hbm.at[wid], i0_v)
        pltpu.sync_copy(c1_hbm.at[wid], i1_v)
        pltpu.sync_copy(w0_hbm.at[wid], w0_v)
        pltpu.sync_copy(w1_hbm.at[wid], w1_v)
        pltpu.sync_copy(g_hbm.at[wid], g_v)
        for j in range(NCH):
            a0 = pltpu.async_copy(outb_hbm.at[i0_v.at[j]], r0_v, sem0)
            a1 = pltpu.async_copy(outb_hbm.at[i1_v.at[j]], r1_v, sem1)
            pltpu.sync_copy(xf_hbm.at[pl.ds(base + j * CH, CH)], xf_v)
            a0.wait()
            a1.wait()

            def token(tl, _):
                gt = jnp.full((L,), j * CH + tl, jnp.int32)
                gs = plsc.load_gather(g_v, [gt])
                a = 1.0 - gs
                b0 = gs * plsc.load_gather(w0_v, [gt])
                b1 = gs * plsc.load_gather(w1_v, [gt])
                for gi in range(D // L):
                    sl = pl.ds(gi * L, L)
                    xf_v[tl, sl] = (a * xf_v[tl, sl] + b0 * r0_v[tl, sl]
                                    + b1 * r1_v[tl, sl])
                return 0

            lax.fori_loop(0, CH, token, 0)
            pltpu.sync_copy(xf_v, y_hbm.at[pl.ds(base + j * CH, CH)])

    return body


def _combine(outb, c0, c1, w0, w1, g, xf):
    return _sc_combine()(outb, c0, c1, w0, w1, g, xf)


def kernel(x, wg_w, wg_b, ffn_w1, ffn_b1, ffn_w2, ffn_b2, router_w, ew1, eb1, ew2, eb2):
    x2d = x.reshape(T, D)
    wgr = jnp.pad(jnp.concatenate([wg_w, router_w], axis=1), ((0, 0), (0, 16 - 1 - E)))
    mb = jnp.pad(wg_b[None, :], ((0, 0), (0, 15)))
    b1 = ffn_b1[None, :]
    b2 = ffn_b2[None, :]

    xf, meta = _dense_pass(x2d, ffn_w1, b1, ffn_w2, b2, wgr, mb)
    iout, fout, scal = _route_pass(meta)

    d0 = iout[:, 0].reshape(NW, NCH, CH)
    d1 = iout[:, 1].reshape(NW, NCH, CH)
    c0 = iout[:, 2].reshape(NW, NCH, CH)
    c1 = iout[:, 3].reshape(NW, NCH, CH)
    w0 = fout[:, 0].reshape(NW, TW)
    w1 = fout[:, 1].reshape(NW, TW)
    g = meta[:, 0].reshape(NW, TW)
    counts = scal[1, 1:1 + E].astype(jnp.int32)

    bufp = _dispatch(x2d, d0, d1)
    outb = _expert_pass(counts, bufp, ew1, eb1, ew2, eb2)
    y = _combine(outb, c0, c1, w0, w1, g, xf)

    x_out = y.reshape(B, S, D)
    l_aux = scal[0, 0]
    used = scal[0, 1]
    return (x_out, l_aux, used, jnp.float32(T))


# speedup vs baseline: 1.2380x; 1.0637x over previous
import functools

import jax
import jax.numpy as jnp
from jax import lax
from jax.experimental import pallas as pl
from jax.experimental.pallas import tpu as pltpu
from jax.experimental.pallas import tpu_sc as plsc

B, S, D = 2, 2048, 1024
E, K, DFF = 8, 2, 2048
T = B * S
CAP = T * K * 5 // (E * 4)
NROW = E * CAP
DUMMY = NROW
NBUF = NROW + 8

NC, NS, L = 2, 16, 16
NW = NC * NS
TW = T // NW
CH = 32
NCH = TW // CH

TT = 512
FJ_A = 2
FB_A = DFF // FJ_A
FJ_E = 4
FB_E = DFF // FJ_E
NEG = -1e30


def _dense_body(x_ref, w1_ref, b1_ref, w2_ref, b2_ref, wgr_ref, mb_ref,
                xf_ref, meta_ref):
    j = pl.program_id(1)
    xb = x_ref[...]
    h = jnp.maximum(xb @ w1_ref[...] + b1_ref[...], 0.0)
    part = jax.lax.dot_general(h, w2_ref[...], (((1,), (0,)), ((), ())),
                               preferred_element_type=jnp.float32)

    @pl.when(j == 0)
    def _():
        xf_ref[...] = part + b2_ref[...]
        m = xb @ wgr_ref[...] + mb_ref[...]
        lanes = lax.broadcasted_iota(jnp.int32, m.shape, 1)
        gate = 1.0 / (1.0 + jnp.exp(-m))
        meta_ref[...] = jnp.where(lanes == 0, gate, m)

    @pl.when(j != 0)
    def _():
        xf_ref[...] += part


def _dense_pass(x2d, w1, b1, w2, b2, wgr, mb):
    return pl.pallas_call(
        _dense_body,
        grid=(T // TT, FJ_A),
        in_specs=[
            pl.BlockSpec((TT, D), lambda t, j: (t, 0)),
            pl.BlockSpec((D, FB_A), lambda t, j: (0, j)),
            pl.BlockSpec((1, FB_A), lambda t, j: (0, j)),
            pl.BlockSpec((FB_A, D), lambda t, j: (j, 0)),
            pl.BlockSpec((1, D), lambda t, j: (0, 0)),
            pl.BlockSpec((D, 16), lambda t, j: (0, 0)),
            pl.BlockSpec((1, 16), lambda t, j: (0, 0)),
        ],
        out_specs=[
            pl.BlockSpec((TT, D), lambda t, j: (t, 0)),
            pl.BlockSpec((TT, 16), lambda t, j: (t, 0)),
        ],
        out_shape=[
            jax.ShapeDtypeStruct((T, D), jnp.float32),
            jax.ShapeDtypeStruct((T, 16), jnp.float32),
        ],
    )(x2d, w1, b1, w2, b2, wgr, mb)


def _route_body(meta_ref, iout_ref, fout_ref, scal_ref, oh_ref, cex_ref, cht_ref):
    CHB = 128
    NCHB = T // CHB
    m = meta_ref[...]
    lanes = lax.broadcasted_iota(jnp.int32, m.shape, 1)
    valid = (lanes >= 1) & (lanes <= E)
    ml = jnp.where(valid, m, NEG)
    mx = jnp.max(ml, axis=-1, keepdims=True)
    p = jnp.exp(ml - mx) * valid.astype(jnp.float32)
    p = p / jnp.sum(p, axis=-1, keepdims=True)

    v0 = jnp.max(p, axis=-1, keepdims=True)
    i0 = jnp.min(jnp.where((p == v0) & valid, lanes, 99), axis=-1, keepdims=True)
    p2 = jnp.where(lanes == i0, -1.0, p)
    v1 = jnp.max(p2, axis=-1, keepdims=True)
    i1 = jnp.min(jnp.where((p2 == v1) & valid, lanes, 99), axis=-1, keepdims=True)
    oh_ref[...] = ((lanes == i0) | (lanes == i1)).astype(jnp.float32)

    r2 = lax.broadcasted_iota(jnp.int32, (CHB, CHB), 0)
    c2 = lax.broadcasted_iota(jnp.int32, (CHB, CHB), 1)
    tril_i = (c2 <= r2).astype(jnp.float32)

    def step(c, _):
        w = oh_ref[pl.ds(c * CHB, CHB), :]
        incl = jax.lax.dot_general(tril_i, w, (((1,), (0,)), ((), ())),
                                   preferred_element_type=jnp.float32)
        cex_ref[pl.ds(c * CHB, CHB), :] = incl - w
        cht_ref[pl.ds(c, 1), :] = incl[CHB - 1:CHB, :]
        return 0

    lax.fori_loop(0, NCHB, step, 0, unroll=4)

    r32 = lax.broadcasted_iota(jnp.int32, (NCHB, NCHB), 0)
    c32 = lax.broadcasted_iota(jnp.int32, (NCHB, NCHB), 1)
    tril32 = (c32 < r32).astype(jnp.float32)
    cht = cht_ref[...]
    offs = jax.lax.dot_general(tril32, cht, (((1,), (0,)), ((), ())),
                               preferred_element_type=jnp.float32)
    offs_full = jnp.broadcast_to(offs[:, None, :], (NCHB, CHB, 16)).reshape(T, 16)
    cex = cex_ref[...] + offs_full

    pos0 = jnp.sum(cex * (lanes == i0), axis=-1, keepdims=True)
    pos1 = jnp.sum(cex * (lanes == i1), axis=-1, keepdims=True)
    counts = offs[NCHB - 1:NCHB, :] + cht[NCHB - 1:NCHB, :]
    psum = jnp.sum(p, axis=0, keepdims=True)
    gsum = jnp.sum(m[:, 0:1])

    e0 = i0 - 1
    e1 = i1 - 1
    p0i = pos0.astype(jnp.int32)
    p1i = pos1.astype(jnp.int32)
    k0 = p0i < CAP
    k1 = p1i < CAP
    s0 = e0 * CAP + p0i
    s1 = e1 * CAP + p1i
    d0 = jnp.where(k0, s0, DUMMY)
    d1 = jnp.where(k1, s1, DUMMY)
    c0 = jnp.where(k0, s0, 0)
    c1 = jnp.where(k1, s1, 0)
    w0 = v0 * k0.astype(jnp.float32)
    w1 = v1 * k1.astype(jnp.float32)

    l8 = lax.broadcasted_iota(jnp.int32, (T, 8), 1)
    iout_ref[...] = jnp.where(l8 == 0, d0,
                    jnp.where(l8 == 1, d1,
                    jnp.where(l8 == 2, c0,
                    jnp.where(l8 == 3, c1, 0))))
    fout_ref[...] = jnp.where(l8 == 0, w0, jnp.where(l8 == 1, w1, 0.0))

    me = psum / jnp.float32(T)
    l_aux = jnp.float32(E) * jnp.sum(me * counts / jnp.float32(T))
    sr = lax.broadcasted_iota(jnp.int32, (8, 16), 0)
    sl = lax.broadcasted_iota(jnp.int32, (8, 16), 1)
    counts_b = jnp.broadcast_to(counts, (8, 16))
    scal_ref[...] = jnp.where((sr == 0) & (sl == 0), l_aux,
                    jnp.where((sr == 0) & (sl == 1), gsum,
                    jnp.where(sr == 1, counts_b, 0.0)))


def _route_pass(meta):
    return pl.pallas_call(
        _route_body,
        out_shape=[
            jax.ShapeDtypeStruct((T, 8), jnp.int32),
            jax.ShapeDtypeStruct((T, 8), jnp.float32),
            jax.ShapeDtypeStruct((8, 16), jnp.float32),
        ],
        scratch_shapes=[
            pltpu.VMEM((T, 16), jnp.float32),
            pltpu.VMEM((T, 16), jnp.float32),
            pltpu.VMEM((T // 128, 16), jnp.float32),
        ],
    )(meta)


@functools.cache
def _sc_dispatch():
    mesh = plsc.VectorSubcoreMesh(core_axis_name="c", subcore_axis_name="s",
                                  num_cores=NC, num_subcores=NS)

    @functools.partial(
        pl.kernel,
        out_type=jax.ShapeDtypeStruct((NBUF, D), jnp.float32),
        mesh=mesh,
        scratch_types=[
            pltpu.VMEM((CH, D), jnp.float32),
            pltpu.VMEM((NCH, CH), jnp.int32),
            pltpu.VMEM((NCH, CH), jnp.int32),
            pltpu.SemaphoreType.DMA,
            pltpu.SemaphoreType.DMA,
        ],
        compiler_params=pltpu.CompilerParams(needs_layout_passes=False),
    )
    def body(x_hbm, d0_hbm, d1_hbm, buf_hbm, rows_v, i0_v, i1_v, sem0, sem1):
        wid = lax.axis_index("s") * NC + lax.axis_index("c")
        base = wid * TW
        pltpu.sync_copy(d0_hbm.at[wid], i0_v)
        pltpu.sync_copy(d1_hbm.at[wid], i1_v)
        for j in range(NCH):
            pltpu.sync_copy(x_hbm.at[pl.ds(base + j * CH, CH)], rows_v)
            c0 = pltpu.async_copy(rows_v, buf_hbm.at[i0_v.at[j]], sem0)
            c1 = pltpu.async_copy(rows_v, buf_hbm.at[i1_v.at[j]], sem1)
            c0.wait()
            c1.wait()

    return body


def _dispatch(x2d, d0, d1):
    return _sc_dispatch()(x2d, d0, d1)


def _expert_body(counts_ref, buf_ref, w1_ref, b1_ref, w2_ref, b2_ref, out_ref):
    e = pl.program_id(0)
    j = pl.program_id(1)
    cnt = counts_ref[e]
    riota = lax.broadcasted_iota(jnp.int32, (CAP, 1), 0)
    xm = jnp.where(riota < cnt, buf_ref[...], 0.0)
    h = jnp.maximum(xm @ w1_ref[0] + b1_ref[0], 0.0)
    part = jax.lax.dot_general(h, w2_ref[0], (((1,), (0,)), ((), ())),
                               preferred_element_type=jnp.float32)

    @pl.when(j == 0)
    def _():
        out_ref[...] = part + b2_ref[0]

    @pl.when(j != 0)
    def _():
        out_ref[...] += part


def _expert_pass(counts, bufp, ew1, eb1, ew2, eb2):
    return pl.pallas_call(
        _expert_body,
        grid=(E, FJ_E),
        in_specs=[
            pl.BlockSpec(memory_space=pltpu.SMEM),
            pl.BlockSpec((CAP, D), lambda e, j: (e, 0)),
            pl.BlockSpec((1, D, FB_E), lambda e, j: (e, 0, j)),
            pl.BlockSpec((1, 1, FB_E), lambda e, j: (e, 0, j)),
            pl.BlockSpec((1, FB_E, D), lambda e, j: (e, j, 0)),
            pl.BlockSpec((1, 1, D), lambda e, j: (e, 0, 0)),
        ],
        out_specs=pl.BlockSpec((CAP, D), lambda e, j: (e, 0)),
        out_shape=jax.ShapeDtypeStruct((NROW, D), jnp.float32),
    )(counts, bufp, ew1, eb1[:, None, :], ew2, eb2[:, None, :])


@functools.cache
def _sc_combine():
    mesh = plsc.VectorSubcoreMesh(core_axis_name="c", subcore_axis_name="s",
                                  num_cores=NC, num_subcores=NS)

    @functools.partial(
        pl.kernel,
        out_type=jax.ShapeDtypeStruct((T, D), jnp.float32),
        mesh=mesh,
        scratch_types=[
            pltpu.VMEM((CH, D), jnp.float32),
            pltpu.VMEM((CH, D), jnp.float32),
            pltpu.VMEM((CH, D), jnp.float32),
            pltpu.VMEM((NCH, CH), jnp.int32),
            pltpu.VMEM((NCH, CH), jnp.int32),
            pltpu.VMEM((TW,), jnp.float32),
            pltpu.VMEM((TW,), jnp.float32),
            pltpu.VMEM((TW,), jnp.float32),
            pltpu.SemaphoreType.DMA,
            pltpu.SemaphoreType.DMA,
        ],
        compiler_params=pltpu.CompilerParams(needs_layout_passes=False),
    )
    def body(outb_hbm, c0_hbm, c1_hbm, w0_hbm, w1_hbm, g_hbm, xf_hbm, y_hbm,
             r0_v, r1_v, xf_v, i0_v, i1_v, w0_v, w1_v, g_v, sem0, sem1):
        wid = lax.axis_index("s") * NC + lax.axis_index("c")
        base = wid * TW
        pltpu.sync_copy(c0_hbm.at[wid], i0_v)
        pltpu.sync_copy(c1_hbm.at[wid], i1_v)
        pltpu.sync_copy(w0_hbm.at[wid], w0_v)
        pltpu.sync_copy(w1_hbm.at[wid], w1_v)
        pltpu.sync_copy(g_hbm.at[wid], g_v)
        for j in range(NCH):
            a0 = pltpu.async_copy(outb_hbm.at[i0_v.at[j]], r0_v, sem0)
            a1 = pltpu.async_copy(outb_hbm.at[i1_v.at[j]], r1_v, sem1)
            pltpu.sync_copy(xf_hbm.at[pl.ds(base + j * CH, CH)], xf_v)
            a0.wait()
            a1.wait()

            def token(tl, _):
                gt = jnp.full((L,), j * CH + tl, jnp.int32)
                gs = plsc.load_gather(g_v, [gt])
                a = 1.0 - gs
                b0 = gs * plsc.load_gather(w0_v, [gt])
                b1 = gs * plsc.load_gather(w1_v, [gt])
                for gi in range(D // L):
                    sl = pl.ds(gi * L, L)
                    xf_v[tl, sl] = (a * xf_v[tl, sl] + b0 * r0_v[tl, sl]
                                    + b1 * r1_v[tl, sl])
                return 0

            lax.fori_loop(0, CH, token, 0)
            pltpu.sync_copy(xf_v, y_hbm.at[pl.ds(base + j * CH, CH)])

    return body


def _combine(outb, c0, c1, w0, w1, g, xf):
    return _sc_combine()(outb, c0, c1, w0, w1, g, xf)


def kernel(x, wg_w, wg_b, ffn_w1, ffn_b1, ffn_w2, ffn_b2, router_w, ew1, eb1, ew2, eb2):
    x2d = x.reshape(T, D)
    wgr = jnp.pad(jnp.concatenate([wg_w, router_w], axis=1), ((0, 0), (0, 16 - 1 - E)))
    mb = jnp.pad(wg_b[None, :], ((0, 0), (0, 15)))
    b1 = ffn_b1[None, :]
    b2 = ffn_b2[None, :]

    xf, meta = _dense_pass(x2d, ffn_w1, b1, ffn_w2, b2, wgr, mb)
    iout, fout, scal = _route_pass(meta)

    d0 = iout[:, 0].reshape(NW, NCH, CH)
    d1 = iout[:, 1].reshape(NW, NCH, CH)
    c0 = iout[:, 2].reshape(NW, NCH, CH)
    c1 = iout[:, 3].reshape(NW, NCH, CH)
    w0 = fout[:, 0].reshape(NW, TW)
    w1 = fout[:, 1].reshape(NW, TW)
    g = meta[:, 0].reshape(NW, TW)
    counts = scal[1, 1:1 + E].astype(jnp.int32)

    bufp = _dispatch(x2d, d0, d1)
    outb = _expert_pass(counts, bufp, ew1, eb1, ew2, eb2)
    y = _combine(outb, c0, c1, w0, w1, g, xf)

    x_out = y.reshape(B, S, D)
    l_aux = scal[0, 0]
    used = scal[0, 1]
    return (x_out, l_aux, used, jnp.float32(T))
